# baseline (device time: 13100 ns/iter reference)
import functools

import jax
import jax.numpy as jnp
from jax import lax
from jax.experimental import pallas as pl
from jax.experimental.pallas import tpu as pltpu

N_CHUNKS = 4


def kernel(x):
    m, n = x.shape
    half = m // 2
    r = half // N_CHUNKS

    def body(x_ref, out_ref, ys_send, ys_recv, xs_send, xs_recv):
        my_x = lax.axis_index("x")
        my_y = lax.axis_index("y")
        y_peer = (my_x, 1 - my_y)
        x_peer = (1 - my_x, my_y)

        barrier_sem = pltpu.get_barrier_semaphore()
        for peer in (y_peer, x_peer):
            pl.semaphore_signal(
                barrier_sem, inc=1, device_id=peer,
                device_id_type=pl.DeviceIdType.MESH,
            )
        pl.semaphore_wait(barrier_sem, 2)

        own_base = my_y * m
        opp_base = (1 - my_y) * m
        send_off = own_base + my_x * half
        recv_off = opp_base + my_x * half
        xrecv_off = opp_base + (1 - my_x) * half

        y_sends = []
        for c in range(N_CHUNKS):
            rows = pl.ds(send_off + c * r, r)
            out_ref[rows, :] = x_ref[
                pl.ds(my_x * half + c * r, r), :
            ].astype(out_ref.dtype)
            rdma = pltpu.make_async_remote_copy(
                src_ref=out_ref.at[rows, :],
                dst_ref=out_ref.at[rows, :],
                send_sem=ys_send.at[c],
                recv_sem=ys_recv.at[c],
                device_id=y_peer,
                device_id_type=pl.DeviceIdType.MESH,
            )
            rdma.start()
            y_sends.append(rdma)

        out_ref[pl.ds(own_base + (1 - my_x) * half, half), :] = x_ref[
            pl.ds((1 - my_x) * half, half), :
        ].astype(out_ref.dtype)

        x_sends = []
        for c in range(N_CHUNKS):
            rows = pl.ds(recv_off + c * r, r)
            recv = pltpu.make_async_remote_copy(
                src_ref=out_ref.at[rows, :],
                dst_ref=out_ref.at[rows, :],
                send_sem=ys_send.at[c],
                recv_sem=ys_recv.at[c],
                device_id=y_peer,
                device_id_type=pl.DeviceIdType.MESH,
            )
            recv.wait_recv()
            fwd = pltpu.make_async_remote_copy(
                src_ref=out_ref.at[rows, :],
                dst_ref=out_ref.at[rows, :],
                send_sem=xs_send.at[c],
                recv_sem=xs_recv.at[c],
                device_id=x_peer,
                device_id_type=pl.DeviceIdType.MESH,
            )
            fwd.start()
            x_sends.append(fwd)

        for c in range(N_CHUNKS):
            rows = pl.ds(xrecv_off + c * r, r)
            recv = pltpu.make_async_remote_copy(
                src_ref=out_ref.at[rows, :],
                dst_ref=out_ref.at[rows, :],
                send_sem=xs_send.at[c],
                recv_sem=xs_recv.at[c],
                device_id=x_peer,
                device_id_type=pl.DeviceIdType.MESH,
            )
            recv.wait_recv()

        for rdma in y_sends:
            rdma.wait_send()
        for rdma in x_sends:
            rdma.wait_send()

        @functools.partial(
            pl.run_scoped, second_barrier=pltpu.SemaphoreType.REGULAR
        )
        def _(second_barrier):
            for peer in (y_peer, x_peer):
                pl.semaphore_signal(
                    second_barrier, inc=1, device_id=peer,
                    device_id_type=pl.DeviceIdType.MESH,
                )
            pl.semaphore_wait(second_barrier, 2)

    return pl.pallas_call(
        body,
        out_shape=jax.ShapeDtypeStruct((2 * m, n), jnp.bfloat16),
        in_specs=[pl.BlockSpec(memory_space=pltpu.VMEM)],
        out_specs=pl.BlockSpec(memory_space=pltpu.VMEM),
        scratch_shapes=[
            pltpu.SemaphoreType.DMA((N_CHUNKS,)),
            pltpu.SemaphoreType.DMA((N_CHUNKS,)),
            pltpu.SemaphoreType.DMA((N_CHUNKS,)),
            pltpu.SemaphoreType.DMA((N_CHUNKS,)),
        ],
        compiler_params=pltpu.CompilerParams(collective_id=0),
    )(x)


# device time: 11946 ns/iter; 1.0966x vs baseline; 1.0966x over previous
import functools

import jax
import jax.numpy as jnp
from jax import lax
from jax.experimental import pallas as pl
from jax.experimental.pallas import tpu as pltpu

N_CHUNKS = 4


def kernel(x):
    m, n = x.shape
    half = m // 2
    r = half // N_CHUNKS

    def body(x_ref, out_ref, ys_send, ys_recv, xs_send, xs_recv):
        my_x = lax.axis_index("x")
        my_y = lax.axis_index("y")
        y_peer = (my_x, 1 - my_y)
        x_peer = (1 - my_x, my_y)

        barrier_sem = pltpu.get_barrier_semaphore()
        for peer in (y_peer, x_peer):
            pl.semaphore_signal(
                barrier_sem, inc=1, device_id=peer,
                device_id_type=pl.DeviceIdType.MESH,
            )
        pl.semaphore_wait(barrier_sem, 2)

        own_base = my_y * m
        opp_base = (1 - my_y) * m
        send_off = own_base + my_x * half
        recv_off = opp_base + my_x * half
        xrecv_off = opp_base + (1 - my_x) * half

        y_sends = []
        for c in range(N_CHUNKS):
            rows = pl.ds(send_off + c * r, r)
            out_ref[rows, :] = x_ref[
                pl.ds(my_x * half + c * r, r), :
            ].astype(out_ref.dtype)
            rdma = pltpu.make_async_remote_copy(
                src_ref=out_ref.at[rows, :],
                dst_ref=out_ref.at[rows, :],
                send_sem=ys_send.at[c],
                recv_sem=ys_recv.at[c],
                device_id=y_peer,
                device_id_type=pl.DeviceIdType.MESH,
            )
            rdma.start()
            y_sends.append(rdma)

        out_ref[pl.ds(own_base + (1 - my_x) * half, half), :] = x_ref[
            pl.ds((1 - my_x) * half, half), :
        ].astype(out_ref.dtype)

        x_sends = []
        for c in range(N_CHUNKS):
            rows = pl.ds(recv_off + c * r, r)
            recv = pltpu.make_async_remote_copy(
                src_ref=out_ref.at[rows, :],
                dst_ref=out_ref.at[rows, :],
                send_sem=ys_send.at[c],
                recv_sem=ys_recv.at[c],
                device_id=y_peer,
                device_id_type=pl.DeviceIdType.MESH,
            )
            recv.wait_recv()
            fwd = pltpu.make_async_remote_copy(
                src_ref=out_ref.at[rows, :],
                dst_ref=out_ref.at[rows, :],
                send_sem=xs_send.at[c],
                recv_sem=xs_recv.at[c],
                device_id=x_peer,
                device_id_type=pl.DeviceIdType.MESH,
            )
            fwd.start()
            x_sends.append(fwd)

        for c in range(N_CHUNKS):
            rows = pl.ds(xrecv_off + c * r, r)
            recv = pltpu.make_async_remote_copy(
                src_ref=out_ref.at[rows, :],
                dst_ref=out_ref.at[rows, :],
                send_sem=xs_send.at[c],
                recv_sem=xs_recv.at[c],
                device_id=x_peer,
                device_id_type=pl.DeviceIdType.MESH,
            )
            recv.wait_recv()

        for rdma in y_sends:
            rdma.wait_send()
        for rdma in x_sends:
            rdma.wait_send()


    return pl.pallas_call(
        body,
        out_shape=jax.ShapeDtypeStruct((2 * m, n), jnp.bfloat16),
        in_specs=[pl.BlockSpec(memory_space=pltpu.VMEM)],
        out_specs=pl.BlockSpec(memory_space=pltpu.VMEM),
        scratch_shapes=[
            pltpu.SemaphoreType.DMA((N_CHUNKS,)),
            pltpu.SemaphoreType.DMA((N_CHUNKS,)),
            pltpu.SemaphoreType.DMA((N_CHUNKS,)),
            pltpu.SemaphoreType.DMA((N_CHUNKS,)),
        ],
        compiler_params=pltpu.CompilerParams(collective_id=0),
    )(x)


# device time: 11471 ns/iter; 1.1420x vs baseline; 1.0414x over previous
import jax
import jax.numpy as jnp
from jax import lax
from jax.experimental import pallas as pl
from jax.experimental.pallas import tpu as pltpu


def kernel(x):
    m, n = x.shape

    def body(x_ref, out_ref, comm_ref, send_sem, recv_sem, copy_sem):
        my_x = lax.axis_index("x")
        my_y = lax.axis_index("y")
        peer = (my_x, 1 - my_y)

        barrier_sem = pltpu.get_barrier_semaphore()
        pl.semaphore_signal(
            barrier_sem, inc=1, device_id=peer,
            device_id_type=pl.DeviceIdType.MESH,
        )
        pl.semaphore_wait(barrier_sem, 1)

        comm_ref[...] = x_ref[...].astype(comm_ref.dtype)
        send = pltpu.make_async_remote_copy(
            src_ref=comm_ref,
            dst_ref=out_ref.at[pl.ds(my_y * m, m), :],
            send_sem=send_sem,
            recv_sem=recv_sem,
            device_id=peer,
            device_id_type=pl.DeviceIdType.MESH,
        )
        send.start()

        local = pltpu.make_async_copy(
            comm_ref, out_ref.at[pl.ds(my_y * m, m), :], copy_sem
        )
        local.start()

        recv = pltpu.make_async_remote_copy(
            src_ref=comm_ref,
            dst_ref=out_ref.at[pl.ds((1 - my_y) * m, m), :],
            send_sem=send_sem,
            recv_sem=recv_sem,
            device_id=peer,
            device_id_type=pl.DeviceIdType.MESH,
        )
        local.wait()
        send.wait_send()
        recv.wait_recv()

    return pl.pallas_call(
        body,
        out_shape=jax.ShapeDtypeStruct((2 * m, n), jnp.bfloat16),
        in_specs=[pl.BlockSpec(memory_space=pltpu.VMEM)],
        out_specs=pl.BlockSpec(memory_space=pl.ANY),
        scratch_shapes=[
            pltpu.VMEM((m, n), jnp.bfloat16),
            pltpu.SemaphoreType.DMA,
            pltpu.SemaphoreType.DMA,
            pltpu.SemaphoreType.DMA,
        ],
        compiler_params=pltpu.CompilerParams(collective_id=0),
    )(x)
